# Initial kernel scaffold; baseline (speedup 1.0000x reference)
import functools

import jax
import jax.numpy as jnp
from jax import lax
from jax.experimental import pallas as pl
from jax.experimental.pallas import tpu as pltpu
from jax.experimental.pallas import tpu_sc as plsc

N = 10000
E = 160000
D = 128
G = 3 * D
NC, NS, L = 2, 16, 16
NW = NC * NS
K = 128
NCHUNK = E // K
TSTEPS = (NCHUNK + NW - 1) // NW
RPS = N // NS

_mesh = plsc.VectorSubcoreMesh(
    core_axis_name="c", subcore_axis_name="s", num_cores=NC, num_subcores=NS)


def _make_sc_agg(with_deg):
  out_type = [jax.ShapeDtypeStruct((NC, N, D), jnp.float32)]
  scratch = [
      pltpu.VMEM((K,), jnp.int32),
      pltpu.VMEM((K,), jnp.int32),
      pltpu.VMEM((K, D), jnp.float32),
      pltpu.VMEM_SHARED((N, D), jnp.float32),
      pltpu.SemaphoreType.DMA,
  ]
  if with_deg:
    out_type.append(jax.ShapeDtypeStruct((NC, N, L), jnp.float32))
    scratch.append(pltpu.VMEM((K, L), jnp.float32))
    scratch.append(pltpu.VMEM_SHARED((N, L), jnp.float32))

  @functools.partial(pl.kernel, out_type=tuple(out_type), mesh=_mesh,
                     scratch_types=scratch)
  def sc_agg(a_hbm, src_hbm, dst_hbm, *refs):
    if with_deg:
      s_out, deg_out, src_v, dst_v, rows_v, s_sh, sem, ones_v, deg_sh = refs
    else:
      s_out, src_v, dst_v, rows_v, s_sh, sem = refs
    cid = lax.axis_index("c")
    sid = lax.axis_index("s")
    wid = cid * NS + sid

    def zrow(i, _):
      rows_v[i // 8, pl.ds((i % 8) * L, L)] = jnp.zeros((L,), jnp.float32)
      return 0
    lax.fori_loop(0, K * D // L, zrow, 0)
    base = sid * RPS
    for j in range(RPS // K):
      pltpu.sync_copy(rows_v, s_sh.at[pl.ds(base + j * K, K)])
    rem = RPS - (RPS // K) * K
    pltpu.sync_copy(rows_v.at[pl.ds(0, rem)],
                    s_sh.at[pl.ds(base + (RPS // K) * K, rem)])

    if with_deg:
      def fill16(i, val):
        ones_v[i, :] = val
        return val
      lax.fori_loop(0, K, fill16, jnp.zeros((L,), jnp.float32))
      for j in range(RPS // K):
        pltpu.sync_copy(ones_v, deg_sh.at[pl.ds(base + j * K, K)])
      pltpu.sync_copy(ones_v.at[pl.ds(0, rem)],
                      deg_sh.at[pl.ds(base + (RPS // K) * K, rem)])
      lax.fori_loop(0, K, fill16, jnp.ones((L,), jnp.float32))

    plsc.subcore_barrier()

    def chunk(t, _):
      cidx = wid + t * NW

      @pl.when(cidx < NCHUNK)
      def _():
        ebase = pl.multiple_of(cidx * K, K)
        pltpu.sync_copy(src_hbm.at[pl.ds(ebase, K)], src_v)
        pltpu.sync_copy(dst_hbm.at[pl.ds(ebase, K)], dst_v)
        pltpu.async_copy(a_hbm.at[src_v], rows_v, sem).wait()
        pltpu.sync_copy(rows_v, s_sh.at[dst_v], add=True)
        if with_deg:
          pltpu.sync_copy(ones_v, deg_sh.at[dst_v], add=True)
      return 0
    lax.fori_loop(0, TSTEPS, chunk, 0)

    plsc.subcore_barrier()
    pltpu.sync_copy(s_sh.at[pl.ds(base, RPS)], s_out.at[cid, pl.ds(base, RPS)])
    if with_deg:
      pltpu.sync_copy(deg_sh.at[pl.ds(base, RPS)],
                      deg_out.at[cid, pl.ds(base, RPS)])

  return sc_agg


_sc_agg_deg = _make_sc_agg(True)
_sc_agg = _make_sc_agg(False)


R = 2000
_f32 = jnp.float32


def _dot(a, b):
  return jnp.dot(a, b, preferred_element_type=_f32)


def _prep_body(x_ref, h_ref, w1, w2, w3, w4, bm, a_ref, b_ref):
  x = x_ref[...]
  h = h_ref[...]
  a_ref[...] = _dot(x, w1[...]) + _dot(h, w2[...])
  b_ref[...] = _dot(x, w3[...]) + _dot(h, w4[...]) + bm[...]


def _row_spec(shape):
  nd = len(shape)
  return pl.BlockSpec(shape, lambda i: (i,) + (0,) * (nd - 1))


def _full_spec(shape):
  nd = len(shape)
  return pl.BlockSpec(shape, lambda i: (0,) * nd)


def _prep(x, h, w1, w2, w3, w4, bm):
  return pl.pallas_call(
      _prep_body,
      grid=(N // R,),
      in_specs=[_row_spec((R, D)), _row_spec((R, D))] +
               [_full_spec((D, D))] * 4 + [_full_spec((1, D))],
      out_specs=[_row_spec((R, D)), _row_spec((R, D))],
      out_shape=[jax.ShapeDtypeStruct((N, D), _f32)] * 2,
  )(x, h, w1, w2, w3, w4, bm)


def _gru_math(x, h, s2, d16, b, wih, whh, bih, bhh):
  s = s2[0] + s2[1]
  deg = d16[0, :, 0:1] + d16[1, :, 0:1]
  denom = jnp.maximum(deg, 1.0)
  mask = (deg > 0.0).astype(_f32)
  c = s / denom + mask * b
  gi = _dot(x, wih[0:D]) + _dot(c, wih[D:2 * D]) + bih
  gh = _dot(h, whh) + bhh
  r = jax.nn.sigmoid(gi[:, 0:D] + gh[:, 0:D])
  z = jax.nn.sigmoid(gi[:, D:2 * D] + gh[:, D:2 * D])
  n = jnp.tanh(gi[:, 2 * D:G] + r * gh[:, 2 * D:G])
  return (1.0 - z) * n + z * h


def _gru_next_body(x_ref, h_ref, s2_ref, d16_ref, b_ref, wih, whh, bih, bhh,
                   w1, w2, w3, w4, bm, hn_ref, an_ref, bn_ref):
  x = x_ref[...]
  hn = _gru_math(x, h_ref[...], s2_ref[...], d16_ref[...], b_ref[...],
                 wih[...], whh[...], bih[...], bhh[...])
  hn_ref[...] = hn
  an_ref[...] = _dot(x, w1[...]) + _dot(hn, w2[...])
  bn_ref[...] = _dot(x, w3[...]) + _dot(hn, w4[...]) + bm[...]


def _gru_last_body(x_ref, h_ref, s2_ref, d16_ref, b_ref, wih, whh, bih, bhh,
                   hn_ref):
  hn_ref[...] = _gru_math(x_ref[...], h_ref[...], s2_ref[...], d16_ref[...],
                          b_ref[...], wih[...], whh[...], bih[...], bhh[...])


def _gru_common_specs():
  return [
      _row_spec((R, D)), _row_spec((R, D)),
      pl.BlockSpec((NC, R, D), lambda i: (0, i, 0)),
      pl.BlockSpec((NC, R, L), lambda i: (0, i, 0)),
      _row_spec((R, D)),
      _full_spec((2 * D, G)), _full_spec((D, G)),
      _full_spec((1, G)), _full_spec((1, G)),
  ]


def _gru_next(x, h, s2, d16, b, wih, whh, bih, bhh, w1, w2, w3, w4, bm):
  return pl.pallas_call(
      _gru_next_body,
      grid=(N // R,),
      in_specs=_gru_common_specs() + [_full_spec((D, D))] * 4 +
               [_full_spec((1, D))],
      out_specs=[_row_spec((R, D))] * 3,
      out_shape=[jax.ShapeDtypeStruct((N, D), _f32)] * 3,
  )(x, h, s2, d16, b, wih, whh, bih, bhh, w1, w2, w3, w4, bm)


def _gru_last(x, h, s2, d16, b, wih, whh, bih, bhh):
  return pl.pallas_call(
      _gru_last_body,
      grid=(N // R,),
      in_specs=_gru_common_specs(),
      out_specs=_row_spec((R, D)),
      out_shape=jax.ShapeDtypeStruct((N, D), _f32),
  )(x, h, s2, d16, b, wih, whh, bih, bhh)


def kernel(x, h, edge_index, W_msg, b_msg, W_ih, W_hh, b_ih, b_hh):
  src = edge_index[0].astype(jnp.int32)
  dst = edge_index[1].astype(jnp.int32)
  wt = W_msg.T
  w1, w2, w3, w4 = wt[0:D], wt[D:2 * D], wt[2 * D:3 * D], wt[3 * D:4 * D]
  bm = b_msg.reshape(1, D)
  wih = W_ih.T
  whh = W_hh.T
  bih = b_ih.reshape(1, G)
  bhh = b_hh.reshape(1, G)

  a1, b1 = _prep(x, h, w1, w2, w3, w4, bm)
  s2, d16 = _sc_agg_deg(a1, src, dst)
  h1, a2, b2 = _gru_next(x, h, s2, d16, b1, wih, whh, bih, bhh,
                         w1, w2, w3, w4, bm)
  s2b, = _sc_agg(a2, src, dst)
  h2 = _gru_last(x, h1, s2b, d16, b2, wih, whh, bih, bhh)
  return h2


# speedup vs baseline: 8.3870x; 8.3870x over previous
import functools

import jax
import jax.numpy as jnp
from jax import lax
from jax.experimental import pallas as pl
from jax.experimental.pallas import tpu as pltpu
from jax.experimental.pallas import tpu_sc as plsc

N = 10000
E = 160000
D = 128
G = 3 * D
NC, NS, L = 2, 16, 16
NW = NC * NS
K = 128
NCHUNK = E // K
TSTEPS = (NCHUNK + NW - 1) // NW
NP = 10240
RPS = NP // NS

@functools.lru_cache(maxsize=None)
def _make_sc_agg(mode):
  mesh = plsc.VectorSubcoreMesh(
      core_axis_name="c", subcore_axis_name="s", num_cores=NC, num_subcores=NS)
  out_type = (jax.ShapeDtypeStruct((NC, NP, D), jnp.float32),)
  scratch = [
      pltpu.VMEM((K,), jnp.int32),
      pltpu.VMEM((K, D), jnp.float32),
      pltpu.VMEM_SHARED((NP, D), jnp.float32),
  ]
  if mode == "sum":
    scratch.append(pltpu.VMEM((K,), jnp.int32))
    scratch.append(pltpu.SemaphoreType.DMA)

  @functools.partial(pl.kernel, out_type=out_type, mesh=mesh,
                     scratch_types=scratch)
  def sc_agg(*args):
    if mode == "sum":
      a_hbm, src_hbm, dst_hbm, s_out, dst_v, rows_v, s_sh, src_v, sem = args
    else:
      dst_hbm, s_out, dst_v, rows_v, s_sh = args
    cid = lax.axis_index("c")
    sid = lax.axis_index("s")
    wid = cid * NS + sid

    def fill(val):
      def body(i, _):
        rows_v[i // 8, pl.ds((i % 8) * L, L)] = jnp.full((L,), val, jnp.float32)
        return 0
      lax.fori_loop(0, K * D // L, body, 0)
    fill(0.0)
    base = sid * RPS
    for j in range(RPS // K):
      pltpu.sync_copy(rows_v, s_sh.at[pl.ds(base + j * K, K)])
    if mode == "deg":
      fill(1.0)
    plsc.subcore_barrier()

    def chunk(t, _):
      cidx = wid + t * NW

      @pl.when(cidx < NCHUNK)
      def _():
        ebase = pl.multiple_of(cidx * K, K)
        pltpu.sync_copy(dst_hbm.at[pl.ds(ebase, K)], dst_v)
        if mode == "sum":
          pltpu.sync_copy(src_hbm.at[pl.ds(ebase, K)], src_v)
          pltpu.async_copy(a_hbm.at[src_v], rows_v, sem).wait()
        pltpu.sync_copy(rows_v, s_sh.at[dst_v], add=True)
      return 0
    lax.fori_loop(0, TSTEPS, chunk, 0)

    plsc.subcore_barrier()
    pltpu.sync_copy(s_sh.at[pl.ds(base, RPS)], s_out.at[cid, pl.ds(base, RPS)])

  return sc_agg


R = 2000
_f32 = jnp.float32


def _dot(a, b):
  return jnp.dot(a, b, preferred_element_type=_f32)


def _prep_body(x_ref, h_ref, w1, w2, w3, w4, bm, a_ref, b_ref):
  x = x_ref[...]
  h = h_ref[...]
  a_ref[...] = _dot(x, w1[...]) + _dot(h, w2[...])
  b_ref[...] = _dot(x, w3[...]) + _dot(h, w4[...]) + bm[...]


def _row_spec(shape):
  nd = len(shape)
  return pl.BlockSpec(shape, lambda i: (i,) + (0,) * (nd - 1))


def _full_spec(shape):
  nd = len(shape)
  return pl.BlockSpec(shape, lambda i: (0,) * nd)


def _prep(x, h, w1, w2, w3, w4, bm):
  return pl.pallas_call(
      _prep_body,
      grid=(N // R,),
      in_specs=[_row_spec((R, D)), _row_spec((R, D))] +
               [_full_spec((D, D))] * 4 + [_full_spec((1, D))],
      out_specs=[_row_spec((R, D)), _row_spec((R, D))],
      out_shape=[jax.ShapeDtypeStruct((N, D), _f32)] * 2,
  )(x, h, w1, w2, w3, w4, bm)


def _gru_math(x, h, s2, d16, b, wih, whh, bih, bhh):
  s = s2[0] + s2[1]
  deg = d16[0, :, 0:1] + d16[1, :, 0:1]
  denom = jnp.maximum(deg, 1.0)
  mask = (deg > 0.0).astype(_f32)
  c = s / denom + mask * b
  gi = _dot(x, wih[0:D]) + _dot(c, wih[D:2 * D]) + bih
  gh = _dot(h, whh) + bhh
  r = jax.nn.sigmoid(gi[:, 0:D] + gh[:, 0:D])
  z = jax.nn.sigmoid(gi[:, D:2 * D] + gh[:, D:2 * D])
  n = jnp.tanh(gi[:, 2 * D:G] + r * gh[:, 2 * D:G])
  return (1.0 - z) * n + z * h


def _gru_next_body(x_ref, h_ref, s2_ref, d16_ref, b_ref, wih, whh, bih, bhh,
                   w1, w2, w3, w4, bm, hn_ref, an_ref, bn_ref):
  x = x_ref[...]
  hn = _gru_math(x, h_ref[...], s2_ref[...], d16_ref[...], b_ref[...],
                 wih[...], whh[...], bih[...], bhh[...])
  hn_ref[...] = hn
  an_ref[...] = _dot(x, w1[...]) + _dot(hn, w2[...])
  bn_ref[...] = _dot(x, w3[...]) + _dot(hn, w4[...]) + bm[...]


def _gru_last_body(x_ref, h_ref, s2_ref, d16_ref, b_ref, wih, whh, bih, bhh,
                   hn_ref):
  hn_ref[...] = _gru_math(x_ref[...], h_ref[...], s2_ref[...], d16_ref[...],
                          b_ref[...], wih[...], whh[...], bih[...], bhh[...])


def _gru_common_specs():
  return [
      _row_spec((R, D)), _row_spec((R, D)),
      pl.BlockSpec((NC, R, D), lambda i: (0, i, 0)),
      pl.BlockSpec((NC, R, D), lambda i: (0, i, 0)),
      _row_spec((R, D)),
      _full_spec((2 * D, G)), _full_spec((D, G)),
      _full_spec((1, G)), _full_spec((1, G)),
  ]


def _gru_next(x, h, s2, d16, b, wih, whh, bih, bhh, w1, w2, w3, w4, bm):
  return pl.pallas_call(
      _gru_next_body,
      grid=(N // R,),
      in_specs=_gru_common_specs() + [_full_spec((D, D))] * 4 +
               [_full_spec((1, D))],
      out_specs=[_row_spec((R, D))] * 3,
      out_shape=[jax.ShapeDtypeStruct((N, D), _f32)] * 3,
  )(x, h, s2, d16, b, wih, whh, bih, bhh, w1, w2, w3, w4, bm)


def _gru_last(x, h, s2, d16, b, wih, whh, bih, bhh):
  return pl.pallas_call(
      _gru_last_body,
      grid=(N // R,),
      in_specs=_gru_common_specs(),
      out_specs=_row_spec((R, D)),
      out_shape=jax.ShapeDtypeStruct((N, D), _f32),
  )(x, h, s2, d16, b, wih, whh, bih, bhh)


def kernel(x, h, edge_index, W_msg, b_msg, W_ih, W_hh, b_ih, b_hh):
  src = edge_index[0].astype(jnp.int32)
  dst = edge_index[1].astype(jnp.int32)
  wt = W_msg.T
  w1, w2, w3, w4 = wt[0:D], wt[D:2 * D], wt[2 * D:3 * D], wt[3 * D:4 * D]
  bm = b_msg.reshape(1, D)
  wih = W_ih.T
  whh = W_hh.T
  bih = b_ih.reshape(1, G)
  bhh = b_hh.reshape(1, G)

  a1, b1 = _prep(x, h, w1, w2, w3, w4, bm)
  d16, = _make_sc_agg("deg")(dst)
  s2, = _make_sc_agg("sum")(a1, src, dst)
  h1, a2, b2 = _gru_next(x, h, s2, d16, b1, wih, whh, bih, bhh,
                         w1, w2, w3, w4, bm)
  s2b, = _make_sc_agg("sum")(a2, src, dst)
  h2 = _gru_last(x, h1, s2b, d16, b2, wih, whh, bih, bhh)
  return h2


# speedup vs baseline: 11.0287x; 1.3150x over previous
import functools

import jax
import jax.numpy as jnp
from jax import lax
from jax.experimental import pallas as pl
from jax.experimental.pallas import tpu as pltpu
from jax.experimental.pallas import tpu_sc as plsc

N = 10000
E = 160000
D = 128
G = 3 * D
NC, NS, L = 2, 16, 16
NW = NC * NS
K = 128
NCHUNK = E // K
TSTEPS = (NCHUNK + NW - 1) // NW
NP = 10240
RPS = NP // NS

@functools.lru_cache(maxsize=None)
def _make_sc_agg(mode):
  mesh = plsc.VectorSubcoreMesh(
      core_axis_name="c", subcore_axis_name="s", num_cores=NC, num_subcores=NS)
  out_type = (jax.ShapeDtypeStruct((NC, NP, D), jnp.float32),)
  if mode == "sum":
    scratch = [
        pltpu.VMEM((K,), jnp.int32), pltpu.VMEM((K,), jnp.int32),
        pltpu.VMEM((K, D), jnp.float32), pltpu.VMEM((K, D), jnp.float32),
        pltpu.VMEM((K,), jnp.int32), pltpu.VMEM((K,), jnp.int32),
        pltpu.SemaphoreType.DMA, pltpu.SemaphoreType.DMA,
        pltpu.VMEM_SHARED((NP, D), jnp.float32),
    ]
  else:
    scratch = [
        pltpu.VMEM((K,), jnp.int32),
        pltpu.VMEM((K, D), jnp.float32),
        pltpu.VMEM_SHARED((NP, D), jnp.float32),
    ]

  @functools.partial(pl.kernel, out_type=out_type, mesh=mesh,
                     scratch_types=scratch)
  def sc_agg(*args):
    if mode == "sum":
      (a_hbm, src_hbm, dst_hbm, s_out,
       dst_v0, dst_v1, rows_v0, rows_v1, src_v0, src_v1,
       sem0, sem1, s_sh) = args
      dst_b = (dst_v0, dst_v1)
      rows_b = (rows_v0, rows_v1)
      src_b = (src_v0, src_v1)
      sem_b = (sem0, sem1)
    else:
      dst_hbm, s_out, dst_v0, rows_v0, s_sh = args
    cid = lax.axis_index("c")
    sid = lax.axis_index("s")
    wid = cid * NS + sid

    def fill(ref, val):
      def body(i, _):
        ref[i // 8, pl.ds((i % 8) * L, L)] = jnp.full((L,), val, jnp.float32)
        return 0
      lax.fori_loop(0, K * D // L, body, 0)
    fill(rows_v0, 0.0)
    base = sid * RPS
    for j in range(RPS // K):
      pltpu.sync_copy(rows_v0, s_sh.at[pl.ds(base + j * K, K)])
    if mode == "deg":
      fill(rows_v0, 1.0)
    plsc.subcore_barrier()

    if mode == "sum":
      def issue(t, b):
        cidx = wid + t * NW

        @pl.when(cidx < NCHUNK)
        def _():
          ebase = pl.multiple_of(cidx * K, K)
          pltpu.sync_copy(src_hbm.at[pl.ds(ebase, K)], src_b[b])
          pltpu.sync_copy(dst_hbm.at[pl.ds(ebase, K)], dst_b[b])
          pltpu.async_copy(a_hbm.at[src_b[b]], rows_b[b], sem_b[b])

      def drain(t, b):
        cidx = wid + t * NW

        @pl.when(cidx < NCHUNK)
        def _():
          pltpu.make_async_copy(a_hbm.at[src_b[b]], rows_b[b], sem_b[b]).wait()
          pltpu.sync_copy(rows_b[b], s_sh.at[dst_b[b]], add=True)

      issue(0, 0)

      def pair(p, _):
        t0 = 2 * p
        issue(t0 + 1, 1)
        drain(t0, 0)
        issue(t0 + 2, 0)
        drain(t0 + 1, 1)
        return 0
      lax.fori_loop(0, TSTEPS // 2, pair, 0)
    else:
      def chunk(t, _):
        cidx = wid + t * NW

        @pl.when(cidx < NCHUNK)
        def _():
          ebase = pl.multiple_of(cidx * K, K)
          pltpu.sync_copy(dst_hbm.at[pl.ds(ebase, K)], dst_v0)
          pltpu.sync_copy(rows_v0, s_sh.at[dst_v0], add=True)
        return 0
      lax.fori_loop(0, TSTEPS, chunk, 0)

    plsc.subcore_barrier()
    pltpu.sync_copy(s_sh.at[pl.ds(base, RPS)], s_out.at[cid, pl.ds(base, RPS)])

  return sc_agg


R = 2000
_f32 = jnp.float32


def _dot(a, b):
  return jnp.dot(a, b, preferred_element_type=_f32)


def _prep_body(x_ref, h_ref, w1, w2, w3, w4, bm, a_ref, b_ref):
  x = x_ref[...]
  h = h_ref[...]
  a_ref[...] = _dot(x, w1[...]) + _dot(h, w2[...])
  b_ref[...] = _dot(x, w3[...]) + _dot(h, w4[...]) + bm[...]


def _row_spec(shape):
  nd = len(shape)
  return pl.BlockSpec(shape, lambda i: (i,) + (0,) * (nd - 1))


def _full_spec(shape):
  nd = len(shape)
  return pl.BlockSpec(shape, lambda i: (0,) * nd)


def _prep(x, h, w1, w2, w3, w4, bm):
  return pl.pallas_call(
      _prep_body,
      grid=(N // R,),
      in_specs=[_row_spec((R, D)), _row_spec((R, D))] +
               [_full_spec((D, D))] * 4 + [_full_spec((1, D))],
      out_specs=[_row_spec((R, D)), _row_spec((R, D))],
      out_shape=[jax.ShapeDtypeStruct((N, D), _f32)] * 2,
  )(x, h, w1, w2, w3, w4, bm)


def _gru_math(x, h, s2, d16, b, wih, whh, bih, bhh):
  s = s2[0] + s2[1]
  deg = d16[0, :, 0:1] + d16[1, :, 0:1]
  denom = jnp.maximum(deg, 1.0)
  mask = (deg > 0.0).astype(_f32)
  c = s / denom + mask * b
  gi = _dot(x, wih[0:D]) + _dot(c, wih[D:2 * D]) + bih
  gh = _dot(h, whh) + bhh
  r = jax.nn.sigmoid(gi[:, 0:D] + gh[:, 0:D])
  z = jax.nn.sigmoid(gi[:, D:2 * D] + gh[:, D:2 * D])
  n = jnp.tanh(gi[:, 2 * D:G] + r * gh[:, 2 * D:G])
  return (1.0 - z) * n + z * h


def _gru_next_body(x_ref, h_ref, s2_ref, d16_ref, b_ref, wih, whh, bih, bhh,
                   w1, w2, w3, w4, bm, hn_ref, an_ref, bn_ref):
  x = x_ref[...]
  hn = _gru_math(x, h_ref[...], s2_ref[...], d16_ref[...], b_ref[...],
                 wih[...], whh[...], bih[...], bhh[...])
  hn_ref[...] = hn
  an_ref[...] = _dot(x, w1[...]) + _dot(hn, w2[...])
  bn_ref[...] = _dot(x, w3[...]) + _dot(hn, w4[...]) + bm[...]


def _gru_last_body(x_ref, h_ref, s2_ref, d16_ref, b_ref, wih, whh, bih, bhh,
                   hn_ref):
  hn_ref[...] = _gru_math(x_ref[...], h_ref[...], s2_ref[...], d16_ref[...],
                          b_ref[...], wih[...], whh[...], bih[...], bhh[...])


def _gru_common_specs():
  return [
      _row_spec((R, D)), _row_spec((R, D)),
      pl.BlockSpec((NC, R, D), lambda i: (0, i, 0)),
      pl.BlockSpec((NC, R, D), lambda i: (0, i, 0)),
      _row_spec((R, D)),
      _full_spec((2 * D, G)), _full_spec((D, G)),
      _full_spec((1, G)), _full_spec((1, G)),
  ]


def _gru_next(x, h, s2, d16, b, wih, whh, bih, bhh, w1, w2, w3, w4, bm):
  return pl.pallas_call(
      _gru_next_body,
      grid=(N // R,),
      in_specs=_gru_common_specs() + [_full_spec((D, D))] * 4 +
               [_full_spec((1, D))],
      out_specs=[_row_spec((R, D))] * 3,
      out_shape=[jax.ShapeDtypeStruct((N, D), _f32)] * 3,
  )(x, h, s2, d16, b, wih, whh, bih, bhh, w1, w2, w3, w4, bm)


def _gru_last(x, h, s2, d16, b, wih, whh, bih, bhh):
  return pl.pallas_call(
      _gru_last_body,
      grid=(N // R,),
      in_specs=_gru_common_specs(),
      out_specs=_row_spec((R, D)),
      out_shape=jax.ShapeDtypeStruct((N, D), _f32),
  )(x, h, s2, d16, b, wih, whh, bih, bhh)


def kernel(x, h, edge_index, W_msg, b_msg, W_ih, W_hh, b_ih, b_hh):
  src = edge_index[0].astype(jnp.int32)
  dst = edge_index[1].astype(jnp.int32)
  wt = W_msg.T
  w1, w2, w3, w4 = wt[0:D], wt[D:2 * D], wt[2 * D:3 * D], wt[3 * D:4 * D]
  bm = b_msg.reshape(1, D)
  wih = W_ih.T
  whh = W_hh.T
  bih = b_ih.reshape(1, G)
  bhh = b_hh.reshape(1, G)

  a1, b1 = _prep(x, h, w1, w2, w3, w4, bm)
  d16, = _make_sc_agg("deg")(dst)
  s2, = _make_sc_agg("sum")(a1, src, dst)
  h1, a2, b2 = _gru_next(x, h, s2, d16, b1, wih, whh, bih, bhh,
                         w1, w2, w3, w4, bm)
  s2b, = _make_sc_agg("sum")(a2, src, dst)
  h2 = _gru_last(x, h1, s2b, d16, b2, wih, whh, bih, bhh)
  return h2


# speedup vs baseline: 13.1296x; 1.1905x over previous
import functools

import jax
import jax.numpy as jnp
from jax import lax
from jax.experimental import pallas as pl
from jax.experimental.pallas import tpu as pltpu
from jax.experimental.pallas import tpu_sc as plsc

N = 10000
E = 160000
D = 128
G = 3 * D
NC, NS, L = 2, 16, 16
NW = NC * NS
K = 128
NCHUNK = E // K
TSTEPS = (NCHUNK + NW - 1) // NW
NP = 10240
RPS = NP // NS

@functools.lru_cache(maxsize=None)
def _make_sc_agg(mode):
  mesh = plsc.VectorSubcoreMesh(
      core_axis_name="c", subcore_axis_name="s", num_cores=NC, num_subcores=NS)
  out_type = [jax.ShapeDtypeStruct((NC, NP, D), jnp.float32)]
  scratch = [
      pltpu.VMEM((K,), jnp.int32), pltpu.VMEM((K,), jnp.int32),
      pltpu.VMEM((K, D), jnp.float32), pltpu.VMEM((K, D), jnp.float32),
      pltpu.VMEM((K,), jnp.int32), pltpu.VMEM((K,), jnp.int32),
      pltpu.SemaphoreType.DMA, pltpu.SemaphoreType.DMA,
      pltpu.VMEM_SHARED((NP, D), jnp.float32),
  ]
  params = None
  if mode == "sum_deg":
    out_type.append(jax.ShapeDtypeStruct((NW, NP), jnp.float32))
    scratch.append(pltpu.VMEM((NP,), jnp.float32))
    params = pltpu.CompilerParams(needs_layout_passes=False)

  @functools.partial(pl.kernel, out_type=tuple(out_type), mesh=mesh,
                     scratch_types=scratch, compiler_params=params)
  def sc_agg(*args):
    if mode == "sum_deg":
      (a_hbm, src_hbm, dst_hbm, z_hbm, s_out, deg_out,
       dst_v0, dst_v1, rows_v0, rows_v1, src_v0, src_v1,
       sem0, sem1, s_sh, hist_v) = args
    else:
      (a_hbm, src_hbm, dst_hbm, z_hbm, s_out,
       dst_v0, dst_v1, rows_v0, rows_v1, src_v0, src_v1,
       sem0, sem1, s_sh) = args
    dst_b = (dst_v0, dst_v1)
    rows_b = (rows_v0, rows_v1)
    src_b = (src_v0, src_v1)
    sem_b = (sem0, sem1)
    cid = lax.axis_index("c")
    sid = lax.axis_index("s")
    wid = cid * NS + sid
    base = sid * RPS

    for j in range(RPS // K):
      pltpu.sync_copy(z_hbm, s_sh.at[pl.ds(base + j * K, K)])
    if mode == "sum_deg":
      def zhist(i, _):
        hist_v[pl.ds(i * L, L)] = jnp.zeros((L,), jnp.float32)
        return 0
      lax.fori_loop(0, NP // L, zhist, 0)
    plsc.subcore_barrier()

    ones16 = jnp.ones((L,), jnp.float32)

    def issue(t, b):
      cidx = wid + t * NW

      @pl.when(cidx < NCHUNK)
      def _():
        ebase = pl.multiple_of(cidx * K, K)
        pltpu.sync_copy(src_hbm.at[pl.ds(ebase, K)], src_b[b])
        pltpu.sync_copy(dst_hbm.at[pl.ds(ebase, K)], dst_b[b])
        pltpu.async_copy(a_hbm.at[src_b[b]], rows_b[b], sem_b[b])
        if mode == "sum_deg":
          for j in range(K // L):
            plsc.addupdate_scatter(hist_v, [dst_b[b][pl.ds(j * L, L)]], ones16)

    def drain(t, b):
      cidx = wid + t * NW

      @pl.when(cidx < NCHUNK)
      def _():
        pltpu.make_async_copy(a_hbm.at[src_b[b]], rows_b[b], sem_b[b]).wait()
        pltpu.sync_copy(rows_b[b], s_sh.at[dst_b[b]], add=True)

    issue(0, 0)

    def pair(p, _):
      t0 = 2 * p
      issue(t0 + 1, 1)
      drain(t0, 0)
      issue(t0 + 2, 0)
      drain(t0 + 1, 1)
      return 0
    lax.fori_loop(0, TSTEPS // 2, pair, 0)

    if mode == "sum_deg":
      pltpu.sync_copy(hist_v, deg_out.at[wid])
    plsc.subcore_barrier()
    pltpu.sync_copy(s_sh.at[pl.ds(base, RPS)], s_out.at[cid, pl.ds(base, RPS)])

  return sc_agg


R = 2048
_f32 = jnp.float32


def _dot(a, b):
  return jnp.dot(a, b, preferred_element_type=_f32)


def _prep_body(x_ref, h_ref, w1, w2, w3, w4, bm, a_ref, b_ref):
  x = x_ref[...]
  h = h_ref[...]
  a_ref[...] = _dot(x, w1[...]) + _dot(h, w2[...])
  b_ref[...] = _dot(x, w3[...]) + _dot(h, w4[...]) + bm[...]


def _row_spec(shape):
  nd = len(shape)
  return pl.BlockSpec(shape, lambda i: (i,) + (0,) * (nd - 1))


def _full_spec(shape):
  nd = len(shape)
  return pl.BlockSpec(shape, lambda i: (0,) * nd)


def _prep(x, h, w1, w2, w3, w4, bm):
  return pl.pallas_call(
      _prep_body,
      grid=(pl.cdiv(N, R),),
      in_specs=[_row_spec((R, D)), _row_spec((R, D))] +
               [_full_spec((D, D))] * 4 + [_full_spec((1, D))],
      out_specs=[_row_spec((R, D)), _row_spec((R, D))],
      out_shape=[jax.ShapeDtypeStruct((N, D), _f32)] * 2,
  )(x, h, w1, w2, w3, w4, bm)


def _gru_math(x, h, s2, d16, b, wih, whh, bih, bhh):
  s = s2[0] + s2[1]
  deg = lax.dot_general(d16, jnp.ones((NW, 1), _f32),
                        (((0,), (0,)), ((), ())),
                        preferred_element_type=_f32)
  denom = jnp.maximum(deg, 1.0)
  mask = (deg > 0.0).astype(_f32)
  c = s / denom + mask * b
  gi = _dot(x, wih[0:D]) + _dot(c, wih[D:2 * D]) + bih
  gh = _dot(h, whh) + bhh
  r = jax.nn.sigmoid(gi[:, 0:D] + gh[:, 0:D])
  z = jax.nn.sigmoid(gi[:, D:2 * D] + gh[:, D:2 * D])
  n = jnp.tanh(gi[:, 2 * D:G] + r * gh[:, 2 * D:G])
  return (1.0 - z) * n + z * h


def _gru_next_body(x_ref, h_ref, s2_ref, d16_ref, b_ref, wih, whh, bih, bhh,
                   w1, w2, w3, w4, bm, hn_ref, an_ref, bn_ref):
  x = x_ref[...]
  hn = _gru_math(x, h_ref[...], s2_ref[...], d16_ref[...], b_ref[...],
                 wih[...], whh[...], bih[...], bhh[...])
  hn_ref[...] = hn
  an_ref[...] = _dot(x, w1[...]) + _dot(hn, w2[...])
  bn_ref[...] = _dot(x, w3[...]) + _dot(hn, w4[...]) + bm[...]


def _gru_last_body(x_ref, h_ref, s2_ref, d16_ref, b_ref, wih, whh, bih, bhh,
                   hn_ref):
  hn_ref[...] = _gru_math(x_ref[...], h_ref[...], s2_ref[...], d16_ref[...],
                          b_ref[...], wih[...], whh[...], bih[...], bhh[...])


def _gru_common_specs():
  return [
      _row_spec((R, D)), _row_spec((R, D)),
      pl.BlockSpec((NC, R, D), lambda i: (0, i, 0)),
      pl.BlockSpec((NW, R), lambda i: (0, i)),
      _row_spec((R, D)),
      _full_spec((2 * D, G)), _full_spec((D, G)),
      _full_spec((1, G)), _full_spec((1, G)),
  ]


def _gru_next(x, h, s2, d16, b, wih, whh, bih, bhh, w1, w2, w3, w4, bm):
  return pl.pallas_call(
      _gru_next_body,
      grid=(pl.cdiv(N, R),),
      in_specs=_gru_common_specs() + [_full_spec((D, D))] * 4 +
               [_full_spec((1, D))],
      out_specs=[_row_spec((R, D))] * 3,
      out_shape=[jax.ShapeDtypeStruct((N, D), _f32)] * 3,
  )(x, h, s2, d16, b, wih, whh, bih, bhh, w1, w2, w3, w4, bm)


def _gru_last(x, h, s2, d16, b, wih, whh, bih, bhh):
  return pl.pallas_call(
      _gru_last_body,
      grid=(pl.cdiv(N, R),),
      in_specs=_gru_common_specs(),
      out_specs=_row_spec((R, D)),
      out_shape=jax.ShapeDtypeStruct((N, D), _f32),
  )(x, h, s2, d16, b, wih, whh, bih, bhh)


def kernel(x, h, edge_index, W_msg, b_msg, W_ih, W_hh, b_ih, b_hh):
  src = edge_index[0].astype(jnp.int32)
  dst = edge_index[1].astype(jnp.int32)
  wt = W_msg.T
  w1, w2, w3, w4 = wt[0:D], wt[D:2 * D], wt[2 * D:3 * D], wt[3 * D:4 * D]
  bm = b_msg.reshape(1, D)
  wih = W_ih.T
  whh = W_hh.T
  bih = b_ih.reshape(1, G)
  bhh = b_hh.reshape(1, G)

  a1, b1 = _prep(x, h, w1, w2, w3, w4, bm)
  zblk = jnp.zeros((K, D), _f32)
  s2, d16 = _make_sc_agg("sum_deg")(a1, src, dst, zblk)
  h1, a2, b2 = _gru_next(x, h, s2, d16, b1, wih, whh, bih, bhh,
                         w1, w2, w3, w4, bm)
  s2b, = _make_sc_agg("sum")(a2, src, dst, zblk)
  h2 = _gru_last(x, h1, s2b, d16, b2, wih, whh, bih, bhh)
  return h2


# speedup vs baseline: 15.3774x; 1.1712x over previous
import functools

import jax
import jax.numpy as jnp
from jax import lax
from jax.experimental import pallas as pl
from jax.experimental.pallas import tpu as pltpu
from jax.experimental.pallas import tpu_sc as plsc

N = 10000
E = 160000
D = 128
G = 3 * D
NC, NS, L = 2, 16, 16
NW = NC * NS
K = 128
NCHUNK = E // K
TSTEPS = (NCHUNK + NW - 1) // NW
NP = 10240
RPS = NP // NS

KC = 125
CPW = E // NW // KC
EPW = E // NW


@functools.lru_cache(maxsize=None)
def _make_sc_agg(mode):
  mesh = plsc.VectorSubcoreMesh(
      core_axis_name="c", subcore_axis_name="s", num_cores=NC, num_subcores=NS)
  out_type = [jax.ShapeDtypeStruct((NC, NP, D), jnp.float32)]
  scratch = [
      pltpu.VMEM((CPW, KC), jnp.int32),
      pltpu.VMEM((CPW, KC), jnp.int32),
      pltpu.SemaphoreType.DMA, pltpu.SemaphoreType.DMA,
      pltpu.VMEM_SHARED((NP, D), jnp.float32),
  ]
  params = None
  if mode == "sum_deg":
    out_type.append(jax.ShapeDtypeStruct((NW, NP), jnp.float32))
    params = pltpu.CompilerParams(needs_layout_passes=False)

  @functools.partial(pl.kernel, out_type=tuple(out_type), mesh=mesh,
                     scratch_types=scratch, compiler_params=params)
  def sc_agg(*args):
    if mode == "sum_deg":
      (a_hbm, src2_hbm, dst2_hbm, dst1_hbm, z_hbm, s_out, deg_out,
       sidx, didx, sem0, sem1, s_sh) = args
    else:
      (a_hbm, src2_hbm, dst2_hbm, dst1_hbm, z_hbm, s_out,
       sidx, didx, sem0, sem1, s_sh) = args
    sem_b = (sem0, sem1)
    cid = lax.axis_index("c")
    sid = lax.axis_index("s")
    wid = cid * NS + sid
    base = sid * RPS

    pltpu.sync_copy(src2_hbm.at[wid], sidx)
    pltpu.sync_copy(dst2_hbm.at[wid], didx)
    for j in range(RPS // K):
      pltpu.sync_copy(z_hbm, s_sh.at[pl.ds(base + j * K, K)])
    if mode == "sum_deg":
      ones16 = jnp.ones((L,), jnp.float32)
      nfull = KC // L
      tail = KC - nfull * L
      tmask = lax.iota(jnp.int32, L) >= (L - tail)

      def hist_phase(hist_v):
        def zhist(i, _):
          hist_v[pl.ds(i * L, L)] = jnp.zeros((L,), jnp.float32)
          return 0
        lax.fori_loop(0, NP // L, zhist, 0)

        def hstep(t, _):
          for j in range(nfull):
            plsc.addupdate_scatter(hist_v, [didx[t, pl.ds(j * L, L)]], ones16)
          idx = jnp.where(tmask, didx[t, pl.ds(KC - L, L)], 0)
          plsc.addupdate_scatter(hist_v, [idx], ones16, mask=tmask)
          return 0
        lax.fori_loop(0, CPW, hstep, 0)
        pltpu.sync_copy(hist_v, deg_out.at[wid])
      pl.run_scoped(hist_phase, pltpu.VMEM((NP,), jnp.float32))
    plsc.subcore_barrier()

    def pipe_phase(rows_v0, rows_v1):
      rows_b = (rows_v0, rows_v1)

      def issue(t, b):
        pltpu.async_copy(a_hbm.at[sidx.at[t]], rows_b[b], sem_b[b])

      def drain(t, b):
        pltpu.make_async_copy(a_hbm.at[sidx.at[t]], rows_b[b], sem_b[b]).wait()
        pltpu.sync_copy(rows_b[b], s_sh.at[didx.at[t]], add=True)

      issue(0, 0)

      def pair(p, _):
        t0 = 2 * p
        issue(t0 + 1, 1)
        drain(t0, 0)

        @pl.when(t0 + 2 < CPW)
        def _():
          issue(t0 + 2, 0)
        drain(t0 + 1, 1)
        return 0
      lax.fori_loop(0, CPW // 2, pair, 0)
    pl.run_scoped(pipe_phase, pltpu.VMEM((KC, D), jnp.float32),
                  pltpu.VMEM((KC, D), jnp.float32))

    plsc.subcore_barrier()
    pltpu.sync_copy(s_sh.at[pl.ds(base, RPS)], s_out.at[cid, pl.ds(base, RPS)])

  return sc_agg


R = 2048
_f32 = jnp.float32


def _dot(a, b):
  return jnp.dot(a, b, preferred_element_type=_f32)


def _prep_body(x_ref, h_ref, w1, w2, w3, w4, bm, a_ref, b_ref):
  x = x_ref[...]
  h = h_ref[...]
  a_ref[...] = _dot(x, w1[...]) + _dot(h, w2[...])
  b_ref[...] = _dot(x, w3[...]) + _dot(h, w4[...]) + bm[...]


def _row_spec(shape):
  nd = len(shape)
  return pl.BlockSpec(shape, lambda i: (i,) + (0,) * (nd - 1))


def _full_spec(shape):
  nd = len(shape)
  return pl.BlockSpec(shape, lambda i: (0,) * nd)


def _prep(x, h, w1, w2, w3, w4, bm):
  return pl.pallas_call(
      _prep_body,
      grid=(pl.cdiv(N, R),),
      in_specs=[_row_spec((R, D)), _row_spec((R, D))] +
               [_full_spec((D, D))] * 4 + [_full_spec((1, D))],
      out_specs=[_row_spec((R, D)), _row_spec((R, D))],
      out_shape=[jax.ShapeDtypeStruct((N, D), _f32)] * 2,
  )(x, h, w1, w2, w3, w4, bm)


def _gru_math(x, h, s2, d16, b, wih, whh, bih, bhh):
  s = s2[0] + s2[1]
  deg = lax.dot_general(d16, jnp.ones((NW, 1), _f32),
                        (((0,), (0,)), ((), ())),
                        preferred_element_type=_f32)
  denom = jnp.maximum(deg, 1.0)
  mask = (deg > 0.0).astype(_f32)
  c = s / denom + mask * b
  gi = _dot(x, wih[0:D]) + _dot(c, wih[D:2 * D]) + bih
  gh = _dot(h, whh) + bhh
  r = jax.nn.sigmoid(gi[:, 0:D] + gh[:, 0:D])
  z = jax.nn.sigmoid(gi[:, D:2 * D] + gh[:, D:2 * D])
  n = jnp.tanh(gi[:, 2 * D:G] + r * gh[:, 2 * D:G])
  return (1.0 - z) * n + z * h


def _gru_next_body(x_ref, h_ref, s2_ref, d16_ref, b_ref, wih, whh, bih, bhh,
                   w1, w2, w3, w4, bm, hn_ref, an_ref, bn_ref):
  x = x_ref[...]
  hn = _gru_math(x, h_ref[...], s2_ref[...], d16_ref[...], b_ref[...],
                 wih[...], whh[...], bih[...], bhh[...])
  hn_ref[...] = hn
  an_ref[...] = _dot(x, w1[...]) + _dot(hn, w2[...])
  bn_ref[...] = _dot(x, w3[...]) + _dot(hn, w4[...]) + bm[...]


def _gru_last_body(x_ref, h_ref, s2_ref, d16_ref, b_ref, wih, whh, bih, bhh,
                   hn_ref):
  hn_ref[...] = _gru_math(x_ref[...], h_ref[...], s2_ref[...], d16_ref[...],
                          b_ref[...], wih[...], whh[...], bih[...], bhh[...])


def _gru_common_specs():
  return [
      _row_spec((R, D)), _row_spec((R, D)),
      pl.BlockSpec((NC, R, D), lambda i: (0, i, 0)),
      pl.BlockSpec((NW, R), lambda i: (0, i)),
      _row_spec((R, D)),
      _full_spec((2 * D, G)), _full_spec((D, G)),
      _full_spec((1, G)), _full_spec((1, G)),
  ]


def _gru_next(x, h, s2, d16, b, wih, whh, bih, bhh, w1, w2, w3, w4, bm):
  return pl.pallas_call(
      _gru_next_body,
      grid=(pl.cdiv(N, R),),
      in_specs=_gru_common_specs() + [_full_spec((D, D))] * 4 +
               [_full_spec((1, D))],
      out_specs=[_row_spec((R, D))] * 3,
      out_shape=[jax.ShapeDtypeStruct((N, D), _f32)] * 3,
  )(x, h, s2, d16, b, wih, whh, bih, bhh, w1, w2, w3, w4, bm)


def _gru_last(x, h, s2, d16, b, wih, whh, bih, bhh):
  return pl.pallas_call(
      _gru_last_body,
      grid=(pl.cdiv(N, R),),
      in_specs=_gru_common_specs(),
      out_specs=_row_spec((R, D)),
      out_shape=jax.ShapeDtypeStruct((N, D), _f32),
  )(x, h, s2, d16, b, wih, whh, bih, bhh)


def kernel(x, h, edge_index, W_msg, b_msg, W_ih, W_hh, b_ih, b_hh):
  src = edge_index[0].astype(jnp.int32)
  dst = edge_index[1].astype(jnp.int32)
  wt = W_msg.T
  w1, w2, w3, w4 = wt[0:D], wt[D:2 * D], wt[2 * D:3 * D], wt[3 * D:4 * D]
  bm = b_msg.reshape(1, D)
  wih = W_ih.T
  whh = W_hh.T
  bih = b_ih.reshape(1, G)
  bhh = b_hh.reshape(1, G)

  a1, b1 = _prep(x, h, w1, w2, w3, w4, bm)
  src2 = src.reshape(NW, CPW, KC)
  dst2 = dst.reshape(NW, CPW, KC)
  zblk = jnp.zeros((K, D), _f32)
  s2, d16 = _make_sc_agg("sum_deg")(a1, src2, dst2, dst, zblk)
  h1, a2, b2 = _gru_next(x, h, s2, d16, b1, wih, whh, bih, bhh,
                         w1, w2, w3, w4, bm)
  s2b, = _make_sc_agg("sum")(a2, src2, dst2, dst, zblk)
  h2 = _gru_last(x, h1, s2b, d16, b2, wih, whh, bih, bhh)
  return h2
